# Initial kernel scaffold; baseline (speedup 1.0000x reference)
import functools
import jax
import jax.numpy as jnp
from jax import lax
from jax.experimental import pallas as pl
from jax.experimental.pallas import tpu as pltpu
from jax.experimental.pallas import tpu_sc as plsc

N = 10000
E = 320000
F_IN = 128
H = 256
C = 128

NC = 2
NS = 16
LANES = 16
EP = E // NS
EB = 160
NCHUNK = EP // EB
NSLICE = N // NS


def _make_sc_segsum(Fh, with_deg):
  mesh = plsc.VectorSubcoreMesh(core_axis_name="c", subcore_axis_name="s")
  out_type = [
      jax.ShapeDtypeStruct((NC, N, Fh), jnp.float32),
  ]
  if with_deg:
    out_type.append(jax.ShapeDtypeStruct((N, 16), jnp.float32))
  scratch = [
      pltpu.VMEM_SHARED((N, Fh), jnp.float32),
      pltpu.VMEM((NSLICE, Fh), jnp.float32),
      pltpu.VMEM((EB,), jnp.int32),
      pltpu.VMEM((EB,), jnp.int32),
      pltpu.VMEM((EB,), jnp.int32),
      pltpu.VMEM((EB, Fh), jnp.float32),
      pltpu.SemaphoreType.DMA,
  ]
  if with_deg:
    scratch += [
        pltpu.VMEM_SHARED((N, 16), jnp.float32),
        pltpu.VMEM((NSLICE, 16), jnp.float32),
        pltpu.VMEM((EB, 16), jnp.float32),
    ]

  def body(h2_hbm, src_hbm, dst_hbm, zeros_hbm, ones_hbm, *rest):
    if with_deg:
      (s_out, deg_out, acc_sh, stage_v, sidx_v, didx_v, gidx_v, rows_v, sem,
       degacc_sh, dstage_v, ones_v) = rest
    else:
      (s_out, acc_sh, stage_v, sidx_v, didx_v, gidx_v, rows_v, sem) = rest
    c = lax.axis_index("c")
    s = lax.axis_index("s")
    n0 = s * NSLICE

    pltpu.sync_copy(zeros_hbm.at[pl.ds(n0, NSLICE)], stage_v)
    pltpu.sync_copy(stage_v, acc_sh.at[pl.ds(n0, NSLICE)])
    if with_deg:
      pltpu.sync_copy(zeros_hbm.at[pl.ds(n0, NSLICE), pl.ds(0, 16)], dstage_v)
      pltpu.sync_copy(dstage_v, degacc_sh.at[pl.ds(n0, NSLICE)])
      pltpu.sync_copy(ones_hbm.at[pl.ds(0, EB)], ones_v)
    plsc.subcore_barrier()

    def chunk(k, carry):
      e0 = s * EP + k * EB
      pltpu.sync_copy(src_hbm.at[pl.ds(e0, EB)], sidx_v)
      pltpu.sync_copy(dst_hbm.at[pl.ds(e0, EB)], didx_v)
      for j in range(EB // LANES):
        v = sidx_v[pl.ds(j * LANES, LANES)]
        gidx_v[pl.ds(j * LANES, LANES)] = v * 2 + c
      pltpu.async_copy(h2_hbm.at[gidx_v], rows_v, sem).wait()
      pltpu.sync_copy(rows_v, acc_sh.at[didx_v], add=True)
      if with_deg:
        @pl.when(c == 0)
        def _():
          pltpu.sync_copy(ones_v, degacc_sh.at[didx_v], add=True)
      return carry

    lax.fori_loop(0, NCHUNK, chunk, 0)
    plsc.subcore_barrier()

    pltpu.sync_copy(acc_sh.at[pl.ds(n0, NSLICE)], stage_v)
    pltpu.sync_copy(stage_v, s_out.at[c, pl.ds(n0, NSLICE)])
    if with_deg:
      @pl.when(c == 0)
      def _():
        pltpu.sync_copy(degacc_sh.at[pl.ds(n0, NSLICE)], dstage_v)
        pltpu.sync_copy(dstage_v, deg_out.at[pl.ds(n0, NSLICE)])

  return pl.kernel(body, out_type=tuple(out_type) if with_deg else out_type[0],
                   mesh=mesh, scratch_types=scratch)


_sc_seg0 = _make_sc_segsum(F_IN // 2, True)
_sc_seg1 = _make_sc_segsum(H // 2, False)


RB = 400
NRB = N // RB

def _full(shape):
  return pl.BlockSpec(shape, lambda i: (0, 0))


def _softplus(x):
  return jnp.maximum(x, 0.0) + jnp.log1p(jnp.exp(-jnp.abs(x)))


def _k_log1p(x_ref, o_ref):
  o_ref[...] = jnp.log1p(x_ref[...])


def _tc_log1p(x):
  return pl.pallas_call(
      _k_log1p,
      grid=(NRB,),
      in_specs=[pl.BlockSpec((RB, F_IN), lambda i: (i, 0))],
      out_specs=pl.BlockSpec((RB, F_IN), lambda i: (i, 0)),
      out_shape=jax.ShapeDtypeStruct((N, F_IN), jnp.float32),
  )(x)


def _k_layer0(h_ref, s0a_ref, s0b_ref, deg_ref, ws_ref, wna_ref, wnb_ref,
              b_ref, o_ref):
  inv = 1.0 / jnp.clip(deg_ref[:, 0:1], 1.0, None)
  t = (jnp.dot(h_ref[...], ws_ref[...], preferred_element_type=jnp.float32)
       + jnp.dot(s0a_ref[...] * inv, wna_ref[...],
                 preferred_element_type=jnp.float32)
       + jnp.dot(s0b_ref[...] * inv, wnb_ref[...],
                 preferred_element_type=jnp.float32)
       + b_ref[0:1, :])
  t = jnp.maximum(t, 0.0)
  nrm = jnp.sqrt(jnp.sum(t * t, axis=1, keepdims=True))
  o_ref[...] = t / jnp.maximum(nrm, 1e-12)


def _tc_layer0(h, s0a, s0b, deg16, ws, wna, wnb, b):
  Fh = F_IN // 2
  return pl.pallas_call(
      _k_layer0,
      grid=(NRB,),
      in_specs=[
          pl.BlockSpec((RB, F_IN), lambda i: (i, 0)),
          pl.BlockSpec((RB, Fh), lambda i: (i, 0)),
          pl.BlockSpec((RB, Fh), lambda i: (i, 0)),
          pl.BlockSpec((RB, 16), lambda i: (i, 0)),
          _full((F_IN, H)), _full((Fh, H)), _full((Fh, H)), _full((1, H)),
      ],
      out_specs=pl.BlockSpec((RB, H), lambda i: (i, 0)),
      out_shape=jax.ShapeDtypeStruct((N, H), jnp.float32),
  )(h, s0a, s0b, deg16, ws, wna, wnb, b)


def _k_layer1fc(h1_ref, s1a_ref, s1b_ref, deg_ref, ws_ref, wna_ref, wnb_ref,
                b_ref, fcw_ref, fcb_ref, t_ref, st_ref):
  i = pl.program_id(0)
  inv = 1.0 / jnp.clip(deg_ref[:, 0:1], 1.0, None)
  h2 = (jnp.dot(h1_ref[...], ws_ref[...], preferred_element_type=jnp.float32)
        + jnp.dot(s1a_ref[...] * inv, wna_ref[...],
                  preferred_element_type=jnp.float32)
        + jnp.dot(s1b_ref[...] * inv, wnb_ref[...],
                  preferred_element_type=jnp.float32)
        + b_ref[0:1, :])
  t = jnp.dot(h2, fcw_ref[...], preferred_element_type=jnp.float32) \
      + fcb_ref[0:1, :]
  t_ref[...] = t

  @pl.when(i == 0)
  def _():
    st_ref[...] = jnp.zeros_like(st_ref)
  st_ref[0:1, :] += jnp.sum(t, axis=0, keepdims=True)
  st_ref[1:2, :] += jnp.sum(t * t, axis=0, keepdims=True)


def _tc_layer1fc(h1, s1a, s1b, deg16, ws, wna, wnb, b, fcw, fcb):
  Hh = H // 2
  return pl.pallas_call(
      _k_layer1fc,
      grid=(NRB,),
      in_specs=[
          pl.BlockSpec((RB, H), lambda i: (i, 0)),
          pl.BlockSpec((RB, Hh), lambda i: (i, 0)),
          pl.BlockSpec((RB, Hh), lambda i: (i, 0)),
          pl.BlockSpec((RB, 16), lambda i: (i, 0)),
          _full((H, H)), _full((Hh, H)), _full((Hh, H)), _full((1, H)),
          _full((H, H)), _full((1, H)),
      ],
      out_specs=[
          pl.BlockSpec((RB, H), lambda i: (i, 0)),
          pl.BlockSpec((8, H), lambda i: (0, 0)),
      ],
      out_shape=[
          jax.ShapeDtypeStruct((N, H), jnp.float32),
          jax.ShapeDtypeStruct((8, H), jnp.float32),
      ],
  )(h1, s1a, s1b, deg16, ws, wna, wnb, b, fcw, fcb)


def _bn_apply(t, st_ref, g_ref, b_ref):
  mu = st_ref[0:1, :] / N
  var = st_ref[1:2, :] / N - mu * mu
  return (t - mu) / jnp.sqrt(var + 1e-5) * g_ref[0:1, :] + b_ref[0:1, :]


def _k_encdec(t_ref, st_ref, g_ref, bb_ref, w21_ref, b21_ref, w22_ref,
              b22_ref, dw_ref, db_ref, zl_ref, zs_ref, d1_ref, st2_ref):
  i = pl.program_id(0)
  e = _softplus(jnp.maximum(_bn_apply(t_ref[...], st_ref, g_ref, bb_ref), 0.0))
  zl = jnp.dot(e, w21_ref[...], preferred_element_type=jnp.float32) \
      + b21_ref[0:1, :]
  zl_ref[...] = zl
  zs_ref[...] = jnp.exp(
      jnp.dot(e, w22_ref[...], preferred_element_type=jnp.float32)
      + b22_ref[0:1, :])
  d1 = jnp.dot(zl, dw_ref[...], preferred_element_type=jnp.float32) \
      + db_ref[0:1, :]
  d1_ref[...] = d1

  @pl.when(i == 0)
  def _():
    st2_ref[...] = jnp.zeros_like(st2_ref)
  st2_ref[0:1, :] += jnp.sum(d1, axis=0, keepdims=True)
  st2_ref[1:2, :] += jnp.sum(d1 * d1, axis=0, keepdims=True)


def _tc_encdec(t, st, g, bb, w21, b21, w22, b22, dw, db):
  return pl.pallas_call(
      _k_encdec,
      grid=(NRB,),
      in_specs=[
          pl.BlockSpec((RB, H), lambda i: (i, 0)),
          pl.BlockSpec((8, H), lambda i: (0, 0)),
          _full((1, H)), _full((1, H)),
          _full((H, H)), _full((1, H)),
          _full((H, H)), _full((1, H)),
          _full((H, H)), _full((1, H)),
      ],
      out_specs=[
          pl.BlockSpec((RB, H), lambda i: (i, 0)),
          pl.BlockSpec((RB, H), lambda i: (i, 0)),
          pl.BlockSpec((RB, H), lambda i: (i, 0)),
          pl.BlockSpec((8, H), lambda i: (0, 0)),
      ],
      out_shape=[
          jax.ShapeDtypeStruct((N, H), jnp.float32),
          jax.ShapeDtypeStruct((N, H), jnp.float32),
          jax.ShapeDtypeStruct((N, H), jnp.float32),
          jax.ShapeDtypeStruct((8, H), jnp.float32),
      ],
  )(t, st, g, bb, w21, b21, w22, b22, dw, db)


def _k_head(d1_ref, st2_ref, g_ref, bb_ref, rw_ref, rb_ref, sw_ref, sb_ref,
            rate_ref, shape_ref):
  dd = _softplus(
      jnp.maximum(_bn_apply(d1_ref[...], st2_ref, g_ref, bb_ref), 0.0))
  rate_ref[...] = _softplus(
      jnp.dot(dd, rw_ref[...], preferred_element_type=jnp.float32)
      + rb_ref[0:1, :])
  shape_ref[...] = _softplus(
      jnp.dot(dd, sw_ref[...], preferred_element_type=jnp.float32)
      + sb_ref[0:1, :])


def _tc_head(d1, st2, g, bb, rw, rb, sw, sb):
  return pl.pallas_call(
      _k_head,
      grid=(NRB,),
      in_specs=[
          pl.BlockSpec((RB, H), lambda i: (i, 0)),
          pl.BlockSpec((8, H), lambda i: (0, 0)),
          _full((1, H)), _full((1, H)),
          _full((H, C)), _full((1, C)),
          _full((H, C)), _full((1, C)),
      ],
      out_specs=[
          pl.BlockSpec((RB, C), lambda i: (i, 0)),
          pl.BlockSpec((RB, C), lambda i: (i, 0)),
      ],
      out_shape=[
          jax.ShapeDtypeStruct((N, C), jnp.float32),
          jax.ShapeDtypeStruct((N, C), jnp.float32),
      ],
  )(d1, st2, g, bb, rw, rb, sw, sb)


@jax.jit
def kernel(x, edge_index, W_self0, W_neigh0, b0, W_self1, W_neigh1, b1, fc_W,
           fc_b, bn_g, bn_b, fc21_W, fc21_b, fc22_W, fc22_b, dfc_W, dfc_b,
           dbn_g, dbn_b, rate_W, rate_b, shape_W, shape_b):
  src = edge_index[0].astype(jnp.int32)
  dst = edge_index[1].astype(jnp.int32)
  zeros_nf = jnp.zeros((N, max(F_IN, H) // 2), jnp.float32)
  ones_n16 = jnp.ones((N, 16), jnp.float32)

  h = _tc_log1p(x)

  h_view = h.reshape(2 * N, F_IN // 2)
  s0, deg16 = _sc_seg0(h_view, src, dst, zeros_nf[:, :F_IN // 2], ones_n16)
  h1 = _tc_layer0(h, s0[0], s0[1], deg16,
                  W_self0, W_neigh0[:F_IN // 2], W_neigh0[F_IN // 2:],
                  b0.reshape(1, H))

  h1_view = h1.reshape(2 * N, H // 2)
  s1 = _sc_seg1(h1_view, src, dst, zeros_nf[:, :H // 2], ones_n16)
  t, st = _tc_layer1fc(h1, s1[0], s1[1], deg16,
                       W_self1, W_neigh1[:H // 2], W_neigh1[H // 2:],
                       b1.reshape(1, H), fc_W, fc_b.reshape(1, H))

  z_loc, z_scale, d1, st2 = _tc_encdec(
      t, st, bn_g.reshape(1, H), bn_b.reshape(1, H),
      fc21_W, fc21_b.reshape(1, H), fc22_W, fc22_b.reshape(1, H),
      dfc_W, dfc_b.reshape(1, H))

  rate, shape = _tc_head(d1, st2, dbn_g.reshape(1, H), dbn_b.reshape(1, H),
                         rate_W, rate_b.reshape(1, C),
                         shape_W, shape_b.reshape(1, C))
  return (z_loc, z_scale, rate, shape)


# speedup vs baseline: 1.8417x; 1.8417x over previous
import functools
import jax
import jax.numpy as jnp
from jax import lax
from jax.experimental import pallas as pl
from jax.experimental.pallas import tpu as pltpu
from jax.experimental.pallas import tpu_sc as plsc

N = 10000
E = 320000
F_IN = 128
H = 256
C = 128

NC = 2
NS = 16
LANES = 16
NP = 10240
NSLICE = NP // NS
Hh = H // 2
NH = NP // 2
NACC = NH + 128
ISLICE = NACC // NS
WSLICE = NH // NS
STG = 80

EP = E // NS
EB = 80
NCHUNK = EP // EB

_MESH = plsc.VectorSubcoreMesh(core_axis_name="c", subcore_axis_name="s")


def _zero_spmem_slice(zeros_hbm, stage_v, acc_sh, base, nrows):
  done = 0
  while done < nrows:
    step = min(STG, nrows - done)
    pltpu.sync_copy(zeros_hbm.at[pl.ds(done, step)], stage_v.at[pl.ds(0, step)])
    pltpu.sync_copy(stage_v.at[pl.ds(0, step)],
                    acc_sh.at[pl.ds(base + done, step)])
    done += step


def _writeout_slice(acc_sh, stage_v, out_ref, acc_base, out_base, nrows):
  done = 0
  while done < nrows:
    step = min(STG, nrows - done)
    pltpu.sync_copy(acc_sh.at[pl.ds(acc_base + done, step)],
                    stage_v.at[pl.ds(0, step)])
    pltpu.sync_copy(stage_v.at[pl.ds(0, step)],
                    out_ref.at[pl.ds(out_base + done, step)])
    done += step


def _redirect(didx_v, didx2_v, nbase):
  for j in range(EB // LANES):
    dv = didx_v[pl.ds(j * LANES, LANES)] - nbase
    oob = (dv < 0) | (dv >= NH)
    didx2_v[pl.ds(j * LANES, LANES)] = jnp.where(oob, NH, dv)


def _sc_l0_body(h_hbm, src_hbm, dst_hbm, zeros_hbm, ones_hbm,
                s_out, acc_sh, stage_v, sidx_v, didx_v, didx2_v, rows_v, sem):
  c = lax.axis_index("c")
  s = lax.axis_index("s")
  nbase = c * NH

  for p in range(2):
    _zero_spmem_slice(zeros_hbm, stage_v, acc_sh, s * ISLICE, ISLICE)
    if p == 1:
      pltpu.sync_copy(ones_hbm, stage_v)
    plsc.subcore_barrier()

    def chunk(k, carry):
      e0 = s * EP + k * EB
      pltpu.sync_copy(dst_hbm.at[pl.ds(e0, EB)], didx_v)
      _redirect(didx_v, didx2_v, nbase)
      if p == 0:
        pltpu.sync_copy(src_hbm.at[pl.ds(e0, EB)], sidx_v)
        pltpu.async_copy(h_hbm.at[sidx_v], rows_v, sem).wait()
        pltpu.sync_copy(rows_v, acc_sh.at[didx2_v], add=True)
      else:
        pltpu.sync_copy(stage_v, acc_sh.at[didx2_v], add=True)
      return carry

    lax.fori_loop(0, NCHUNK, chunk, 0)
    plsc.subcore_barrier()

    _writeout_slice(acc_sh, stage_v, s_out.at[p],
                    s * WSLICE, nbase + s * WSLICE, WSLICE)
    plsc.subcore_barrier()


_sc_l0 = pl.kernel(
    _sc_l0_body,
    out_type=jax.ShapeDtypeStruct((2, NP, F_IN), jnp.float32),
    mesh=_MESH,
    scratch_types=[
        pltpu.VMEM_SHARED((NACC, F_IN), jnp.float32),
        pltpu.VMEM((STG, F_IN), jnp.float32),
        pltpu.VMEM((EB,), jnp.int32),
        pltpu.VMEM((EB,), jnp.int32),
        pltpu.VMEM((EB,), jnp.int32),
        pltpu.VMEM((EB, F_IN), jnp.float32),
        pltpu.SemaphoreType.DMA,
    ])


def _sc_l1_body(h2_hbm, src_hbm, dst_hbm, zeros_hbm,
                s_out, acc_sh, stage_v, sidx_v, didx_v, didx2_v, gidx_v,
                rows_v, sem):
  c = lax.axis_index("c")
  s = lax.axis_index("s")
  nbase = c * NH

  for f in range(2):
    _zero_spmem_slice(zeros_hbm, stage_v, acc_sh, s * ISLICE, ISLICE)
    plsc.subcore_barrier()

    def chunk(k, carry):
      e0 = s * EP + k * EB
      pltpu.sync_copy(src_hbm.at[pl.ds(e0, EB)], sidx_v)
      pltpu.sync_copy(dst_hbm.at[pl.ds(e0, EB)], didx_v)
      for j in range(EB // LANES):
        sv = sidx_v[pl.ds(j * LANES, LANES)]
        gidx_v[pl.ds(j * LANES, LANES)] = sv * 2 + f
      _redirect(didx_v, didx2_v, nbase)
      pltpu.async_copy(h2_hbm.at[gidx_v], rows_v, sem).wait()
      pltpu.sync_copy(rows_v, acc_sh.at[didx2_v], add=True)
      return carry

    lax.fori_loop(0, NCHUNK, chunk, 0)
    plsc.subcore_barrier()

    _writeout_slice(acc_sh, stage_v, s_out.at[f],
                    s * WSLICE, nbase + s * WSLICE, WSLICE)
    plsc.subcore_barrier()


_sc_l1 = pl.kernel(
    _sc_l1_body,
    out_type=jax.ShapeDtypeStruct((2, NP, Hh), jnp.float32),
    mesh=_MESH,
    scratch_types=[
        pltpu.VMEM_SHARED((NACC, Hh), jnp.float32),
        pltpu.VMEM((STG, Hh), jnp.float32),
        pltpu.VMEM((EB,), jnp.int32),
        pltpu.VMEM((EB,), jnp.int32),
        pltpu.VMEM((EB,), jnp.int32),
        pltpu.VMEM((EB,), jnp.int32),
        pltpu.VMEM((EB, Hh), jnp.float32),
        pltpu.SemaphoreType.DMA,
    ])


RB = 400
NRB = N // RB

def _full(shape):
  return pl.BlockSpec(shape, lambda i: (0, 0))


def _softplus(x):
  return jnp.maximum(x, 0.0) + jnp.log1p(jnp.exp(-jnp.abs(x)))


def _k_log1p(x_ref, o_ref):
  o_ref[...] = jnp.log1p(x_ref[...])


def _tc_log1p(x):
  return pl.pallas_call(
      _k_log1p,
      grid=(NRB,),
      in_specs=[pl.BlockSpec((RB, F_IN), lambda i: (i, 0))],
      out_specs=pl.BlockSpec((RB, F_IN), lambda i: (i, 0)),
      out_shape=jax.ShapeDtypeStruct((N, F_IN), jnp.float32),
  )(x)


def _k_layer0(h_ref, s0_ref, deg_ref, ws_ref, wn_ref, b_ref, o_ref):
  inv = 1.0 / jnp.clip(deg_ref[:, 0:1], 1.0, None)
  m = s0_ref[...] * inv
  t = (jnp.dot(h_ref[...], ws_ref[...], preferred_element_type=jnp.float32)
       + jnp.dot(m, wn_ref[...], preferred_element_type=jnp.float32)
       + b_ref[0:1, :])
  t = jnp.maximum(t, 0.0)
  nrm = jnp.sqrt(jnp.sum(t * t, axis=1, keepdims=True))
  o_ref[...] = t / jnp.maximum(nrm, 1e-12)


def _tc_layer0(h, s0, deg128, ws, wn, b):
  return pl.pallas_call(
      _k_layer0,
      grid=(NRB,),
      in_specs=[
          pl.BlockSpec((RB, F_IN), lambda i: (i, 0)),
          pl.BlockSpec((RB, F_IN), lambda i: (i, 0)),
          pl.BlockSpec((RB, F_IN), lambda i: (i, 0)),
          _full((F_IN, H)), _full((F_IN, H)), _full((1, H)),
      ],
      out_specs=pl.BlockSpec((RB, H), lambda i: (i, 0)),
      out_shape=jax.ShapeDtypeStruct((N, H), jnp.float32),
  )(h, s0, deg128, ws, wn, b)


def _k_layer1fc(h1_ref, s1a_ref, s1b_ref, deg_ref, ws_ref, wna_ref, wnb_ref,
                b_ref, fcw_ref, fcb_ref, t_ref, st_ref):
  i = pl.program_id(0)
  inv = 1.0 / jnp.clip(deg_ref[:, 0:1], 1.0, None)
  h2 = (jnp.dot(h1_ref[...], ws_ref[...], preferred_element_type=jnp.float32)
        + jnp.dot(s1a_ref[...] * inv, wna_ref[...],
                  preferred_element_type=jnp.float32)
        + jnp.dot(s1b_ref[...] * inv, wnb_ref[...],
                  preferred_element_type=jnp.float32)
        + b_ref[0:1, :])
  t = jnp.dot(h2, fcw_ref[...], preferred_element_type=jnp.float32) \
      + fcb_ref[0:1, :]
  t_ref[...] = t

  @pl.when(i == 0)
  def _():
    st_ref[...] = jnp.zeros_like(st_ref)
  st_ref[0:1, :] += jnp.sum(t, axis=0, keepdims=True)
  st_ref[1:2, :] += jnp.sum(t * t, axis=0, keepdims=True)


def _tc_layer1fc(h1, s1a, s1b, deg16, ws, wna, wnb, b, fcw, fcb):
  Hh = H // 2
  return pl.pallas_call(
      _k_layer1fc,
      grid=(NRB,),
      in_specs=[
          pl.BlockSpec((RB, H), lambda i: (i, 0)),
          pl.BlockSpec((RB, Hh), lambda i: (i, 0)),
          pl.BlockSpec((RB, Hh), lambda i: (i, 0)),
          pl.BlockSpec((RB, F_IN), lambda i: (i, 0)),
          _full((H, H)), _full((Hh, H)), _full((Hh, H)), _full((1, H)),
          _full((H, H)), _full((1, H)),
      ],
      out_specs=[
          pl.BlockSpec((RB, H), lambda i: (i, 0)),
          pl.BlockSpec((8, H), lambda i: (0, 0)),
      ],
      out_shape=[
          jax.ShapeDtypeStruct((N, H), jnp.float32),
          jax.ShapeDtypeStruct((8, H), jnp.float32),
      ],
  )(h1, s1a, s1b, deg16, ws, wna, wnb, b, fcw, fcb)


def _bn_apply(t, st_ref, g_ref, b_ref):
  mu = st_ref[0:1, :] / N
  var = st_ref[1:2, :] / N - mu * mu
  return (t - mu) / jnp.sqrt(var + 1e-5) * g_ref[0:1, :] + b_ref[0:1, :]


def _k_encdec(t_ref, st_ref, g_ref, bb_ref, w21_ref, b21_ref, w22_ref,
              b22_ref, dw_ref, db_ref, zl_ref, zs_ref, d1_ref, st2_ref):
  i = pl.program_id(0)
  e = _softplus(jnp.maximum(_bn_apply(t_ref[...], st_ref, g_ref, bb_ref), 0.0))
  zl = jnp.dot(e, w21_ref[...], preferred_element_type=jnp.float32) \
      + b21_ref[0:1, :]
  zl_ref[...] = zl
  zs_ref[...] = jnp.exp(
      jnp.dot(e, w22_ref[...], preferred_element_type=jnp.float32)
      + b22_ref[0:1, :])
  d1 = jnp.dot(zl, dw_ref[...], preferred_element_type=jnp.float32) \
      + db_ref[0:1, :]
  d1_ref[...] = d1

  @pl.when(i == 0)
  def _():
    st2_ref[...] = jnp.zeros_like(st2_ref)
  st2_ref[0:1, :] += jnp.sum(d1, axis=0, keepdims=True)
  st2_ref[1:2, :] += jnp.sum(d1 * d1, axis=0, keepdims=True)


def _tc_encdec(t, st, g, bb, w21, b21, w22, b22, dw, db):
  return pl.pallas_call(
      _k_encdec,
      grid=(NRB,),
      in_specs=[
          pl.BlockSpec((RB, H), lambda i: (i, 0)),
          pl.BlockSpec((8, H), lambda i: (0, 0)),
          _full((1, H)), _full((1, H)),
          _full((H, H)), _full((1, H)),
          _full((H, H)), _full((1, H)),
          _full((H, H)), _full((1, H)),
      ],
      out_specs=[
          pl.BlockSpec((RB, H), lambda i: (i, 0)),
          pl.BlockSpec((RB, H), lambda i: (i, 0)),
          pl.BlockSpec((RB, H), lambda i: (i, 0)),
          pl.BlockSpec((8, H), lambda i: (0, 0)),
      ],
      out_shape=[
          jax.ShapeDtypeStruct((N, H), jnp.float32),
          jax.ShapeDtypeStruct((N, H), jnp.float32),
          jax.ShapeDtypeStruct((N, H), jnp.float32),
          jax.ShapeDtypeStruct((8, H), jnp.float32),
      ],
  )(t, st, g, bb, w21, b21, w22, b22, dw, db)


def _k_head(d1_ref, st2_ref, g_ref, bb_ref, rw_ref, rb_ref, sw_ref, sb_ref,
            rate_ref, shape_ref):
  dd = _softplus(
      jnp.maximum(_bn_apply(d1_ref[...], st2_ref, g_ref, bb_ref), 0.0))
  rate_ref[...] = _softplus(
      jnp.dot(dd, rw_ref[...], preferred_element_type=jnp.float32)
      + rb_ref[0:1, :])
  shape_ref[...] = _softplus(
      jnp.dot(dd, sw_ref[...], preferred_element_type=jnp.float32)
      + sb_ref[0:1, :])


def _tc_head(d1, st2, g, bb, rw, rb, sw, sb):
  return pl.pallas_call(
      _k_head,
      grid=(NRB,),
      in_specs=[
          pl.BlockSpec((RB, H), lambda i: (i, 0)),
          pl.BlockSpec((8, H), lambda i: (0, 0)),
          _full((1, H)), _full((1, H)),
          _full((H, C)), _full((1, C)),
          _full((H, C)), _full((1, C)),
      ],
      out_specs=[
          pl.BlockSpec((RB, C), lambda i: (i, 0)),
          pl.BlockSpec((RB, C), lambda i: (i, 0)),
      ],
      out_shape=[
          jax.ShapeDtypeStruct((N, C), jnp.float32),
          jax.ShapeDtypeStruct((N, C), jnp.float32),
      ],
  )(d1, st2, g, bb, rw, rb, sw, sb)


@jax.jit
def kernel(x, edge_index, W_self0, W_neigh0, b0, W_self1, W_neigh1, b1, fc_W,
           fc_b, bn_g, bn_b, fc21_W, fc21_b, fc22_W, fc22_b, dfc_W, dfc_b,
           dbn_g, dbn_b, rate_W, rate_b, shape_W, shape_b):
  src = edge_index[0].astype(jnp.int32)
  dst = edge_index[1].astype(jnp.int32)
  zeros_np = jnp.zeros((NP, Hh), jnp.float32)
  cols = lax.broadcasted_iota(jnp.int32, (EB, F_IN), 1)
  ones_eb = jnp.where(cols == 0, 1.0, 0.0).astype(jnp.float32)

  h = _tc_log1p(x)

  s0 = _sc_l0(h, src, dst, zeros_np, ones_eb)
  deg128 = s0[1, :N]
  h1 = _tc_layer0(h, s0[0, :N], deg128,
                  W_self0, W_neigh0, b0.reshape(1, H))

  s1 = _sc_l1(h1.reshape(2 * N, Hh), src, dst, zeros_np)
  t, st = _tc_layer1fc(h1, s1[0, :N], s1[1, :N], deg128,
                       W_self1, W_neigh1[:H // 2], W_neigh1[H // 2:],
                       b1.reshape(1, H), fc_W, fc_b.reshape(1, H))

  z_loc, z_scale, d1, st2 = _tc_encdec(
      t, st, bn_g.reshape(1, H), bn_b.reshape(1, H),
      fc21_W, fc21_b.reshape(1, H), fc22_W, fc22_b.reshape(1, H),
      dfc_W, dfc_b.reshape(1, H))

  rate, shape = _tc_head(d1, st2, dbn_g.reshape(1, H), dbn_b.reshape(1, H),
                         rate_W, rate_b.reshape(1, C),
                         shape_W, shape_b.reshape(1, C))
  return (z_loc, z_scale, rate, shape)


# speedup vs baseline: 3.5760x; 1.9417x over previous
import functools
import jax
import jax.numpy as jnp
from jax import lax
from jax.experimental import pallas as pl
from jax.experimental.pallas import tpu as pltpu
from jax.experimental.pallas import tpu_sc as plsc

N = 10000
E = 320000
F_IN = 128
H = 256
C = 128

NC = 2
NS = 16
LANES = 16
NP = 10240
Hh = H // 2
NH = NP // 2
NACC = NH + 128
ISLICE = NACC // NS
WSLICE = NH // NS

EP = E // NS
EB = 80
NCHUNK = EP // EB
NPAIR = NCHUNK // 2
EBD = 160
NCHD = EP // EBD

_MESH = plsc.VectorSubcoreMesh(core_axis_name="c", subcore_axis_name="s")


def _stage_rows(src_ref, dst_ref, stage_v, src_base, dst_base, nrows):
  done = 0
  while done < nrows:
    step = min(2 * EB, nrows - done)
    pltpu.sync_copy(src_ref.at[pl.ds(src_base + done, step)],
                    stage_v.at[pl.ds(0, step)])
    pltpu.sync_copy(stage_v.at[pl.ds(0, step)],
                    dst_ref.at[pl.ds(dst_base + done, step)])
    done += step


def _agg_pass(table_hbm, src_hbm, dst_hbm, acc_sh, s, nbase, mult, off,
              sidxA, sidxB, didxA, didxB, gidxA, gidxB, d2A, d2B, rows_v,
              isemA, isemB, gsemA, gsemB):
  base = s * EP
  rowsA = rows_v.at[pl.ds(0, EB)]
  rowsB = rows_v.at[pl.ds(EB, EB)]

  def fire_idx(k, sidx_v, didx_v, sem):
    e0 = base + k * EB
    pltpu.async_copy(src_hbm.at[pl.ds(e0, EB)], sidx_v.at[pl.ds(0, EB)], sem)
    pltpu.async_copy(dst_hbm.at[pl.ds(e0, EB)], didx_v.at[pl.ds(0, EB)], sem)

  def wait_idx(k, sidx_v, didx_v, sem):
    e0 = base + k * EB
    pltpu.make_async_copy(src_hbm.at[pl.ds(e0, EB)],
                          sidx_v.at[pl.ds(0, EB)], sem).wait()
    pltpu.make_async_copy(dst_hbm.at[pl.ds(e0, EB)],
                          didx_v.at[pl.ds(0, EB)], sem).wait()

  def transform(sidx_v, didx_v, gidx_v, d2_v):
    for j in range(EB // LANES):
      sv = sidx_v[pl.ds(j * LANES, LANES)]
      gidx_v[pl.ds(j * LANES, LANES)] = sv * mult + off
      dv = didx_v[pl.ds(j * LANES, LANES)] - nbase
      oob = (dv < 0) | (dv >= NH)
      d2_v[pl.ds(j * LANES, LANES)] = jnp.where(oob, NH, dv)

  def fire_gather(gidx_v, rows, sem):
    pltpu.async_copy(table_hbm.at[gidx_v], rows, sem)

  def wait_gather(gidx_v, rows, sem):
    pltpu.make_async_copy(table_hbm.at[gidx_v], rows, sem).wait()

  fire_idx(0, sidxA, didxA, isemA)
  wait_idx(0, sidxA, didxA, isemA)
  transform(sidxA, didxA, gidxA, d2A)
  fire_gather(gidxA, rowsA, gsemA)
  fire_idx(1, sidxB, didxB, isemB)

  def pair(p, carry):
    a = 2 * p
    wait_idx(a + 1, sidxB, didxB, isemB)
    transform(sidxB, didxB, gidxB, d2B)
    fire_gather(gidxB, rowsB, gsemB)

    @pl.when(p < NPAIR - 1)
    def _():
      fire_idx(a + 2, sidxA, didxA, isemA)

    wait_gather(gidxA, rowsA, gsemA)
    pltpu.sync_copy(rowsA, acc_sh.at[d2A], add=True)

    @pl.when(p < NPAIR - 1)
    def _():
      wait_idx(a + 2, sidxA, didxA, isemA)
      transform(sidxA, didxA, gidxA, d2A)
      fire_gather(gidxA, rowsA, gsemA)
      fire_idx(a + 3, sidxB, didxB, isemB)

    wait_gather(gidxB, rowsB, gsemB)
    pltpu.sync_copy(rowsB, acc_sh.at[d2B], add=True)
    return carry

  lax.fori_loop(0, NPAIR, pair, 0)


def _sc_l0_body(h_hbm, src_hbm, dst_hbm, zeros_hbm, ones_hbm,
                s_out, acc_sh, sidxA, sidxB, didxA, didxB, gidxA, gidxB,
                d2A, d2B, d2dA, d2dB, rows_v,
                isemA, isemB, gsemA, gsemB):
  c = lax.axis_index("c")
  s = lax.axis_index("s")
  nbase = c * NH
  base = s * EP

  _stage_rows(zeros_hbm, acc_sh, rows_v, s * ISLICE, s * ISLICE, ISLICE)
  plsc.subcore_barrier()
  _agg_pass(h_hbm, src_hbm, dst_hbm, acc_sh, s, nbase, 1, 0,
            sidxA, sidxB, didxA, didxB, gidxA, gidxB, d2A, d2B, rows_v,
            isemA, isemB, gsemA, gsemB)
  plsc.subcore_barrier()
  _stage_rows(acc_sh, s_out.at[0], rows_v, s * WSLICE, nbase + s * WSLICE,
              WSLICE)
  plsc.subcore_barrier()

  _stage_rows(zeros_hbm, acc_sh, rows_v, s * ISLICE, s * ISLICE, ISLICE)
  pltpu.sync_copy(ones_hbm, rows_v)
  plsc.subcore_barrier()

  def fire_didx(k, didx_v, sem):
    pltpu.async_copy(dst_hbm.at[pl.ds(base + k * EBD, EBD)], didx_v, sem)

  def wait_didx(k, didx_v, sem):
    pltpu.make_async_copy(dst_hbm.at[pl.ds(base + k * EBD, EBD)],
                          didx_v, sem).wait()

  def dtransform(didx_v, d2_v):
    for j in range(EBD // LANES):
      dv = didx_v[pl.ds(j * LANES, LANES)] - nbase
      oob = (dv < 0) | (dv >= NH)
      d2_v[pl.ds(j * LANES, LANES)] = jnp.where(oob, NH, dv)

  fire_didx(0, didxA, isemA)
  fire_didx(1, didxB, isemB)
  wait_didx(0, didxA, isemA)
  dtransform(didxA, d2dA)

  def dpair(p, carry):
    a = 2 * p
    fire_didx(a + 2, didxA, isemA)
    pltpu.sync_copy(rows_v, acc_sh.at[d2dA], add=True)
    wait_didx(a + 1, didxB, isemB)
    dtransform(didxB, d2dB)

    @pl.when(p < NCHD // 2 - 1)
    def _():
      fire_didx(a + 3, didxB, isemB)

    pltpu.sync_copy(rows_v, acc_sh.at[d2dB], add=True)
    wait_didx(a + 2, didxA, isemA)
    dtransform(didxA, d2dA)
    return carry

  lax.fori_loop(0, NCHD // 2, dpair, 0)
  pltpu.sync_copy(rows_v, acc_sh.at[d2dA], add=True)
  plsc.subcore_barrier()
  _stage_rows(acc_sh, s_out.at[1], rows_v, s * WSLICE, nbase + s * WSLICE,
              WSLICE)


_sc_l0 = pl.kernel(
    _sc_l0_body,
    out_type=jax.ShapeDtypeStruct((2, NP, F_IN), jnp.float32),
    mesh=_MESH,
    scratch_types=[
        pltpu.VMEM_SHARED((NACC, F_IN), jnp.float32),
        pltpu.VMEM((EB,), jnp.int32),
        pltpu.VMEM((EB,), jnp.int32),
        pltpu.VMEM((EBD,), jnp.int32),
        pltpu.VMEM((EBD,), jnp.int32),
        pltpu.VMEM((EB,), jnp.int32),
        pltpu.VMEM((EB,), jnp.int32),
        pltpu.VMEM((EB,), jnp.int32),
        pltpu.VMEM((EB,), jnp.int32),
        pltpu.VMEM((EBD,), jnp.int32),
        pltpu.VMEM((EBD,), jnp.int32),
        pltpu.VMEM((2 * EB, F_IN), jnp.float32),
        pltpu.SemaphoreType.DMA,
        pltpu.SemaphoreType.DMA,
        pltpu.SemaphoreType.DMA,
        pltpu.SemaphoreType.DMA,
    ])


def _sc_l1_body(h2_hbm, src_hbm, dst_hbm, zeros_hbm,
                s_out, acc_sh, sidxA, sidxB, didxA, didxB, gidxA, gidxB,
                d2A, d2B, rows_v, isemA, isemB, gsemA, gsemB):
  c = lax.axis_index("c")
  s = lax.axis_index("s")
  nbase = c * NH

  for f in range(2):
    _stage_rows(zeros_hbm, acc_sh, rows_v, s * ISLICE, s * ISLICE, ISLICE)
    plsc.subcore_barrier()
    _agg_pass(h2_hbm, src_hbm, dst_hbm, acc_sh, s, nbase, 2, f,
              sidxA, sidxB, didxA, didxB, gidxA, gidxB, d2A, d2B, rows_v,
              isemA, isemB, gsemA, gsemB)
    plsc.subcore_barrier()
    _stage_rows(acc_sh, s_out.at[f], rows_v, s * WSLICE, nbase + s * WSLICE,
                WSLICE)
    plsc.subcore_barrier()


_sc_l1 = pl.kernel(
    _sc_l1_body,
    out_type=jax.ShapeDtypeStruct((2, NP, Hh), jnp.float32),
    mesh=_MESH,
    scratch_types=[
        pltpu.VMEM_SHARED((NACC, Hh), jnp.float32),
        pltpu.VMEM((EB,), jnp.int32),
        pltpu.VMEM((EB,), jnp.int32),
        pltpu.VMEM((EB,), jnp.int32),
        pltpu.VMEM((EB,), jnp.int32),
        pltpu.VMEM((EB,), jnp.int32),
        pltpu.VMEM((EB,), jnp.int32),
        pltpu.VMEM((EB,), jnp.int32),
        pltpu.VMEM((EB,), jnp.int32),
        pltpu.VMEM((2 * EB, Hh), jnp.float32),
        pltpu.SemaphoreType.DMA,
        pltpu.SemaphoreType.DMA,
        pltpu.SemaphoreType.DMA,
        pltpu.SemaphoreType.DMA,
    ])


RB = 400
NRB = N // RB

def _full(shape):
  return pl.BlockSpec(shape, lambda i: (0, 0))


def _softplus(x):
  return jnp.maximum(x, 0.0) + jnp.log1p(jnp.exp(-jnp.abs(x)))


def _k_log1p(x_ref, o_ref):
  o_ref[...] = jnp.log1p(x_ref[...])


def _tc_log1p(x):
  return pl.pallas_call(
      _k_log1p,
      grid=(NRB,),
      in_specs=[pl.BlockSpec((RB, F_IN), lambda i: (i, 0))],
      out_specs=pl.BlockSpec((RB, F_IN), lambda i: (i, 0)),
      out_shape=jax.ShapeDtypeStruct((N, F_IN), jnp.float32),
  )(x)


def _k_layer0(h_ref, s0_ref, deg_ref, ws_ref, wn_ref, b_ref, o_ref):
  inv = 1.0 / jnp.clip(deg_ref[:, 0:1], 1.0, None)
  m = s0_ref[...] * inv
  t = (jnp.dot(h_ref[...], ws_ref[...], preferred_element_type=jnp.float32)
       + jnp.dot(m, wn_ref[...], preferred_element_type=jnp.float32)
       + b_ref[0:1, :])
  t = jnp.maximum(t, 0.0)
  nrm = jnp.sqrt(jnp.sum(t * t, axis=1, keepdims=True))
  o_ref[...] = t / jnp.maximum(nrm, 1e-12)


def _tc_layer0(h, s0, deg128, ws, wn, b):
  return pl.pallas_call(
      _k_layer0,
      grid=(NRB,),
      in_specs=[
          pl.BlockSpec((RB, F_IN), lambda i: (i, 0)),
          pl.BlockSpec((RB, F_IN), lambda i: (i, 0)),
          pl.BlockSpec((RB, F_IN), lambda i: (i, 0)),
          _full((F_IN, H)), _full((F_IN, H)), _full((1, H)),
      ],
      out_specs=pl.BlockSpec((RB, H), lambda i: (i, 0)),
      out_shape=jax.ShapeDtypeStruct((N, H), jnp.float32),
  )(h, s0, deg128, ws, wn, b)


def _k_layer1fc(h1_ref, s1a_ref, s1b_ref, deg_ref, ws_ref, wna_ref, wnb_ref,
                b_ref, fcw_ref, fcb_ref, t_ref, st_ref):
  i = pl.program_id(0)
  inv = 1.0 / jnp.clip(deg_ref[:, 0:1], 1.0, None)
  h2 = (jnp.dot(h1_ref[...], ws_ref[...], preferred_element_type=jnp.float32)
        + jnp.dot(s1a_ref[...] * inv, wna_ref[...],
                  preferred_element_type=jnp.float32)
        + jnp.dot(s1b_ref[...] * inv, wnb_ref[...],
                  preferred_element_type=jnp.float32)
        + b_ref[0:1, :])
  t = jnp.dot(h2, fcw_ref[...], preferred_element_type=jnp.float32) \
      + fcb_ref[0:1, :]
  t_ref[...] = t

  @pl.when(i == 0)
  def _():
    st_ref[...] = jnp.zeros_like(st_ref)
  st_ref[0:1, :] += jnp.sum(t, axis=0, keepdims=True)
  st_ref[1:2, :] += jnp.sum(t * t, axis=0, keepdims=True)


def _tc_layer1fc(h1, s1a, s1b, deg128, ws, wna, wnb, b, fcw, fcb):
  return pl.pallas_call(
      _k_layer1fc,
      grid=(NRB,),
      in_specs=[
          pl.BlockSpec((RB, H), lambda i: (i, 0)),
          pl.BlockSpec((RB, Hh), lambda i: (i, 0)),
          pl.BlockSpec((RB, Hh), lambda i: (i, 0)),
          pl.BlockSpec((RB, F_IN), lambda i: (i, 0)),
          _full((H, H)), _full((Hh, H)), _full((Hh, H)), _full((1, H)),
          _full((H, H)), _full((1, H)),
      ],
      out_specs=[
          pl.BlockSpec((RB, H), lambda i: (i, 0)),
          pl.BlockSpec((8, H), lambda i: (0, 0)),
      ],
      out_shape=[
          jax.ShapeDtypeStruct((N, H), jnp.float32),
          jax.ShapeDtypeStruct((8, H), jnp.float32),
      ],
  )(h1, s1a, s1b, deg128, ws, wna, wnb, b, fcw, fcb)


def _bn_apply(t, st_ref, g_ref, b_ref):
  mu = st_ref[0:1, :] / N
  var = st_ref[1:2, :] / N - mu * mu
  return (t - mu) / jnp.sqrt(var + 1e-5) * g_ref[0:1, :] + b_ref[0:1, :]


def _k_encdec(t_ref, st_ref, g_ref, bb_ref, w21_ref, b21_ref, w22_ref,
              b22_ref, dw_ref, db_ref, zl_ref, zs_ref, d1_ref, st2_ref):
  i = pl.program_id(0)
  e = _softplus(jnp.maximum(_bn_apply(t_ref[...], st_ref, g_ref, bb_ref), 0.0))
  zl = jnp.dot(e, w21_ref[...], preferred_element_type=jnp.float32) \
      + b21_ref[0:1, :]
  zl_ref[...] = zl
  zs_ref[...] = jnp.exp(
      jnp.dot(e, w22_ref[...], preferred_element_type=jnp.float32)
      + b22_ref[0:1, :])
  d1 = jnp.dot(zl, dw_ref[...], preferred_element_type=jnp.float32) \
      + db_ref[0:1, :]
  d1_ref[...] = d1

  @pl.when(i == 0)
  def _():
    st2_ref[...] = jnp.zeros_like(st2_ref)
  st2_ref[0:1, :] += jnp.sum(d1, axis=0, keepdims=True)
  st2_ref[1:2, :] += jnp.sum(d1 * d1, axis=0, keepdims=True)


def _tc_encdec(t, st, g, bb, w21, b21, w22, b22, dw, db):
  return pl.pallas_call(
      _k_encdec,
      grid=(NRB,),
      in_specs=[
          pl.BlockSpec((RB, H), lambda i: (i, 0)),
          pl.BlockSpec((8, H), lambda i: (0, 0)),
          _full((1, H)), _full((1, H)),
          _full((H, H)), _full((1, H)),
          _full((H, H)), _full((1, H)),
          _full((H, H)), _full((1, H)),
      ],
      out_specs=[
          pl.BlockSpec((RB, H), lambda i: (i, 0)),
          pl.BlockSpec((RB, H), lambda i: (i, 0)),
          pl.BlockSpec((RB, H), lambda i: (i, 0)),
          pl.BlockSpec((8, H), lambda i: (0, 0)),
      ],
      out_shape=[
          jax.ShapeDtypeStruct((N, H), jnp.float32),
          jax.ShapeDtypeStruct((N, H), jnp.float32),
          jax.ShapeDtypeStruct((N, H), jnp.float32),
          jax.ShapeDtypeStruct((8, H), jnp.float32),
      ],
  )(t, st, g, bb, w21, b21, w22, b22, dw, db)


def _k_head(d1_ref, st2_ref, g_ref, bb_ref, rw_ref, rb_ref, sw_ref, sb_ref,
            rate_ref, shape_ref):
  dd = _softplus(
      jnp.maximum(_bn_apply(d1_ref[...], st2_ref, g_ref, bb_ref), 0.0))
  rate_ref[...] = _softplus(
      jnp.dot(dd, rw_ref[...], preferred_element_type=jnp.float32)
      + rb_ref[0:1, :])
  shape_ref[...] = _softplus(
      jnp.dot(dd, sw_ref[...], preferred_element_type=jnp.float32)
      + sb_ref[0:1, :])


def _tc_head(d1, st2, g, bb, rw, rb, sw, sb):
  return pl.pallas_call(
      _k_head,
      grid=(NRB,),
      in_specs=[
          pl.BlockSpec((RB, H), lambda i: (i, 0)),
          pl.BlockSpec((8, H), lambda i: (0, 0)),
          _full((1, H)), _full((1, H)),
          _full((H, C)), _full((1, C)),
          _full((H, C)), _full((1, C)),
      ],
      out_specs=[
          pl.BlockSpec((RB, C), lambda i: (i, 0)),
          pl.BlockSpec((RB, C), lambda i: (i, 0)),
      ],
      out_shape=[
          jax.ShapeDtypeStruct((N, C), jnp.float32),
          jax.ShapeDtypeStruct((N, C), jnp.float32),
      ],
  )(d1, st2, g, bb, rw, rb, sw, sb)


@jax.jit
def kernel(x, edge_index, W_self0, W_neigh0, b0, W_self1, W_neigh1, b1, fc_W,
           fc_b, bn_g, bn_b, fc21_W, fc21_b, fc22_W, fc22_b, dfc_W, dfc_b,
           dbn_g, dbn_b, rate_W, rate_b, shape_W, shape_b):
  src = edge_index[0].astype(jnp.int32)
  dst = edge_index[1].astype(jnp.int32)
  zeros_np = jnp.zeros((NP, Hh), jnp.float32)
  cols = lax.broadcasted_iota(jnp.int32, (2 * EB, F_IN), 1)
  ones_eb = jnp.where(cols == 0, 1.0, 0.0).astype(jnp.float32)

  h = _tc_log1p(x)

  s0 = _sc_l0(h, src, dst, zeros_np, ones_eb)
  deg128 = s0[1, :N]
  h1 = _tc_layer0(h, s0[0, :N], deg128,
                  W_self0, W_neigh0, b0.reshape(1, H))

  s1 = _sc_l1(h1.reshape(2 * N, Hh), src, dst, zeros_np)
  t, st = _tc_layer1fc(h1, s1[0, :N], s1[1, :N], deg128,
                       W_self1, W_neigh1[:H // 2], W_neigh1[H // 2:],
                       b1.reshape(1, H), fc_W, fc_b.reshape(1, H))

  z_loc, z_scale, d1, st2 = _tc_encdec(
      t, st, bn_g.reshape(1, H), bn_b.reshape(1, H),
      fc21_W, fc21_b.reshape(1, H), fc22_W, fc22_b.reshape(1, H),
      dfc_W, dfc_b.reshape(1, H))

  rate, shape = _tc_head(d1, st2, dbn_g.reshape(1, H), dbn_b.reshape(1, H),
                         rate_W, rate_b.reshape(1, C),
                         shape_W, shape_b.reshape(1, C))
  return (z_loc, z_scale, rate, shape)
